# baseline (speedup 1.0000x reference)
import functools

import jax
import jax.numpy as jnp
from jax import lax
from jax.experimental import pallas as pl
from jax.experimental.pallas import tpu as pltpu
from jax.experimental.pallas import tpu_sc as plsc

_NC = 2
_NS = 16
_L = 16
_NW = _NC * _NS
_B = 16384
_BPW = _B // _NW
_F = 64
_CH = 128
_NCH = _BPW // _CH

_mesh = plsc.VectorSubcoreMesh(core_axis_name="c", subcore_axis_name="s")


@functools.partial(
    pl.kernel,
    mesh=_mesh,
    out_type=jax.ShapeDtypeStruct((_B,), jnp.float32),
    compiler_params=pltpu.CompilerParams(
        needs_layout_passes=False, use_tc_tiling_on_sc=False
    ),
    scratch_types=[
        pltpu.VMEM((_NCH, _CH), jnp.int32),
        pltpu.VMEM((_NCH, _CH), jnp.int32),
        pltpu.VMEM((_NCH, _CH), jnp.int32),
        pltpu.VMEM((_NCH, _CH), jnp.int32),
        pltpu.VMEM((_BPW,), jnp.int32),
        pltpu.VMEM((_BPW,), jnp.int32),
        pltpu.VMEM((_BPW, _F), jnp.float32),
        pltpu.VMEM((_BPW, _F), jnp.float32),
        pltpu.VMEM((_BPW, _L), jnp.float32),
        pltpu.VMEM((_BPW, _L), jnp.float32),
        pltpu.VMEM((_L,), jnp.float32),
        pltpu.VMEM((_L, _L), jnp.float32),
        pltpu.VMEM((_BPW,), jnp.float32),
        pltpu.SemaphoreType.DMA,
    ],
)
def _funk_sc(uids_hbm, iids_hbm, ufac_hbm, ifac_hbm, ubias_hbm, ibias_hbm,
             gb_hbm, out_hbm, uidx, iidx, ubidx, ibidx, ulo, ilo, urows,
             irows, ubrows, ibrows, gbv, tile, outv, sem):
    c = lax.axis_index("c")
    s = lax.axis_index("s")
    wid = s * _NC + c
    base = wid * _BPW

    pltpu.sync_copy(uids_hbm.at[pl.ds(wid * _NCH, _NCH)], uidx)
    pltpu.sync_copy(iids_hbm.at[pl.ds(wid * _NCH, _NCH)], iidx)
    pltpu.sync_copy(gb_hbm, gbv)

    copies = []
    for j in range(_NCH):
        dst = pl.ds(j * _CH, _CH)
        copies.append(pltpu.async_copy(ufac_hbm.at[uidx.at[j]], urows.at[dst], sem))
        copies.append(pltpu.async_copy(ifac_hbm.at[iidx.at[j]], irows.at[dst], sem))

    for j in range(_NCH):
        for t in range(_CH // _L):
            sl = pl.ds(t * _L, _L)
            fl = pl.ds(j * _CH + t * _L, _L)
            uv = uidx[j, sl]
            iv = iidx[j, sl]
            ubidx[j, sl] = jnp.right_shift(uv, 4)
            ibidx[j, sl] = jnp.right_shift(iv, 4)
            ulo[fl] = jnp.bitwise_and(uv, 15)
            ilo[fl] = jnp.bitwise_and(iv, 15)

    for j in range(_NCH):
        dst = pl.ds(j * _CH, _CH)
        copies.append(pltpu.async_copy(ubias_hbm.at[ubidx.at[j]], ubrows.at[dst], sem))
        copies.append(pltpu.async_copy(ibias_hbm.at[ibidx.at[j]], ibrows.at[dst], sem))
    for cp in copies:
        cp.wait()

    lane = lax.iota(jnp.int32, _L)
    zeros = jnp.zeros((_L,), jnp.int32)
    gb = gbv[...]

    def group_body(g, carry):
        row0 = g * _L
        for r in range(_L):
            row = row0 + r
            acc = urows[row, pl.ds(0, _L)] * irows[row, pl.ds(0, _L)]
            for k in range(1, _F // _L):
                acc = acc + urows[row, pl.ds(k * _L, _L)] * irows[row, pl.ds(k * _L, _L)]
            tile[r, ...] = acc
        ssum = plsc.load_gather(tile, [lane, zeros])
        for j in range(1, _L):
            ssum = ssum + plsc.load_gather(tile, [lane, jnp.full((_L,), j, jnp.int32)])
        rows16 = row0 + lane
        ub = plsc.load_gather(ubrows, [rows16, ulo[pl.ds(row0, _L)]])
        ib = plsc.load_gather(ibrows, [rows16, ilo[pl.ds(row0, _L)]])
        outv[pl.ds(row0, _L)] = ssum + ub + ib + gb
        return carry

    lax.fori_loop(0, _BPW // _L, group_body, 0)

    pltpu.sync_copy(outv, out_hbm.at[pl.ds(base, _BPW)])


def kernel(user_ids, item_ids, user_factors, item_factors, user_bias,
           item_bias, global_bias):
    uids2 = user_ids.reshape(_B // _CH, _CH)
    iids2 = item_ids.reshape(_B // _CH, _CH)
    ub2 = user_bias.reshape(user_bias.shape[0] // _L, _L)
    ib2 = item_bias.reshape(item_bias.shape[0] // _L, _L)
    gb16 = jnp.broadcast_to(global_bias.astype(jnp.float32).reshape(()), (_L,))
    return _funk_sc(uids2, iids2, user_factors, item_factors, ub2, ib2, gb16)
